# Initial kernel scaffold; baseline (speedup 1.0000x reference)
import functools

import jax
import jax.numpy as jnp
from jax import lax
from jax.experimental import pallas as pl
from jax.experimental.pallas import tpu as pltpu
from jax.experimental.pallas import tpu_sc as plsc

N_RAW = 10000
E_RAW = 320000
F = 128
N_P = 10240
E_P = 327680
NC, NS = 2, 16
NW = NC * NS
E_TILE = E_P // NW
CHUNK = 128
N_CHUNKS = E_TILE // CHUNK
ROWS_TILE = N_P // NS

_mesh = plsc.VectorSubcoreMesh(
    core_axis_name="c", subcore_axis_name="s", num_cores=NC, num_subcores=NS
)


def _zero_fill(ref, n_rows, n_cols):
    zero = jnp.zeros((16,), jnp.float32)

    def body(i, carry):
        for k in range(n_cols // 16):
            ref[i, pl.ds(k * 16, 16)] = zero
        return carry

    lax.fori_loop(0, n_rows, body, 0)


@functools.partial(
    pl.kernel,
    out_type=jax.ShapeDtypeStruct((NC, 2, N_P, 16), jnp.float32),
    mesh=_mesh,
    scratch_types=[
        pltpu.VMEM((N_CHUNKS, CHUNK), jnp.int32),
        pltpu.VMEM((N_CHUNKS, CHUNK), jnp.int32),
        pltpu.VMEM((CHUNK, 16), jnp.float32),
        pltpu.VMEM_SHARED((N_P, 16), jnp.float32),
        pltpu.VMEM_SHARED((N_P, 16), jnp.float32),
    ],
)
def _deg_kernel(src_hbm, dst_hbm, out_hbm, idx_s, idx_d, ones_v, scnt, dcnt):
    c = lax.axis_index("c")
    s = lax.axis_index("s")
    wid = c * NS + s
    pltpu.sync_copy(src_hbm.at[wid], idx_s)
    pltpu.sync_copy(dst_hbm.at[wid], idx_d)
    _zero_fill(ones_v, CHUNK, 16)
    for r in range(ROWS_TILE // CHUNK):
        base = s * ROWS_TILE + r * CHUNK
        pltpu.sync_copy(ones_v, scnt.at[pl.ds(base, CHUNK)])
        pltpu.sync_copy(ones_v, dcnt.at[pl.ds(base, CHUNK)])
    plsc.subcore_barrier()
    one = jnp.ones((16,), jnp.float32)

    def fill(i, carry):
        ones_v[i] = one
        return carry

    lax.fori_loop(0, CHUNK, fill, 0)

    def chunk(j, carry):
        pltpu.sync_copy(ones_v, scnt.at[idx_s.at[j]], add=True)
        pltpu.sync_copy(ones_v, dcnt.at[idx_d.at[j]], add=True)
        return carry

    lax.fori_loop(0, N_CHUNKS, chunk, 0)
    plsc.subcore_barrier()
    pltpu.sync_copy(scnt.at[pl.ds(s * ROWS_TILE, ROWS_TILE)],
                    out_hbm.at[c, 0, pl.ds(s * ROWS_TILE, ROWS_TILE)])
    pltpu.sync_copy(dcnt.at[pl.ds(s * ROWS_TILE, ROWS_TILE)],
                    out_hbm.at[c, 1, pl.ds(s * ROWS_TILE, ROWS_TILE)])


@functools.partial(
    pl.kernel,
    out_type=jax.ShapeDtypeStruct((NC, N_P, F), jnp.float32),
    mesh=_mesh,
    scratch_types=[
        pltpu.VMEM((N_CHUNKS, CHUNK), jnp.int32),
        pltpu.VMEM((N_CHUNKS, CHUNK), jnp.int32),
        pltpu.VMEM((CHUNK, F), jnp.float32),
        pltpu.VMEM_SHARED((N_P, F), jnp.float32),
    ],
)
def _prop_kernel(table_hbm, src_hbm, dst_hbm, out_hbm, idx_s, idx_d, rows_v, agg):
    c = lax.axis_index("c")
    s = lax.axis_index("s")
    wid = c * NS + s
    pltpu.sync_copy(src_hbm.at[wid], idx_s)
    pltpu.sync_copy(dst_hbm.at[wid], idx_d)
    _zero_fill(rows_v, CHUNK, F)
    for r in range(ROWS_TILE // CHUNK):
        base = s * ROWS_TILE + r * CHUNK
        pltpu.sync_copy(rows_v, agg.at[pl.ds(base, CHUNK)])
    plsc.subcore_barrier()

    def chunk(j, carry):
        pltpu.sync_copy(table_hbm.at[idx_s.at[j]], rows_v)
        pltpu.sync_copy(rows_v, agg.at[idx_d.at[j]], add=True)
        return carry

    lax.fori_loop(0, N_CHUNKS, chunk, 0)
    plsc.subcore_barrier()
    pltpu.sync_copy(agg.at[pl.ds(s * ROWS_TILE, ROWS_TILE)],
                    out_hbm.at[c, pl.ds(s * ROWS_TILE, ROWS_TILE)])


def _deg_inv(cnt_pair, col):
    cnt = cnt_pair[0] + cnt_pair[1]
    return lax.rsqrt(jnp.maximum(cnt[:, col:col + 1], 1.0))


def _tc_first_body(feat_ref, cs_ref, w_ref, out_ref):
    dsrc = _deg_inv(cs_ref[...], 0)
    out_ref[...] = jnp.dot(feat_ref[...] * dsrc, w_ref[...],
                           preferred_element_type=jnp.float32)


def _tc_mid_body(p_ref, cs_ref, cd_ref, b_ref, w_ref, out_ref):
    p = p_ref[0] + p_ref[1]
    ddst = _deg_inv(cd_ref[...], 0)
    h = jnp.maximum(p * ddst + b_ref[...], 0.0)
    dsrc = _deg_inv(cs_ref[...], 0)
    out_ref[...] = jnp.dot(h * dsrc, w_ref[...],
                           preferred_element_type=jnp.float32)


def _tc_last_body(p_ref, cd_ref, b_ref, out_ref):
    p = p_ref[0] + p_ref[1]
    ddst = _deg_inv(cd_ref[...], 0)
    out_ref[...] = p * ddst + b_ref[...]


_BLK = 1024
_GRID = N_P // _BLK

_feat_spec = pl.BlockSpec((_BLK, F), lambda i: (i, 0))
_cnt_spec = pl.BlockSpec((2, _BLK, 16), lambda i: (0, i, 0))
_p_spec = pl.BlockSpec((2, _BLK, F), lambda i: (0, i, 0))
_w_spec = pl.BlockSpec((F, F), lambda i: (0, 0))
_b_spec = pl.BlockSpec((1, F), lambda i: (0, 0))
_out_spec = pl.BlockSpec((_BLK, F), lambda i: (i, 0))
_out_shape = jax.ShapeDtypeStruct((N_P, F), jnp.float32)

_tc_first = pl.pallas_call(
    _tc_first_body, grid=(_GRID,),
    in_specs=[_feat_spec, _cnt_spec, _w_spec],
    out_specs=_out_spec, out_shape=_out_shape)

_tc_mid = pl.pallas_call(
    _tc_mid_body, grid=(_GRID,),
    in_specs=[_p_spec, _cnt_spec, _cnt_spec, _b_spec, _w_spec],
    out_specs=_out_spec, out_shape=_out_shape)

_tc_last = pl.pallas_call(
    _tc_last_body, grid=(_GRID,),
    in_specs=[_p_spec, _cnt_spec, _b_spec],
    out_specs=_out_spec, out_shape=_out_shape)


def kernel(features, edge_index, W0, b0, W1, b1, W2, b2):
    feat_p = jnp.pad(features, ((0, N_P - N_RAW), (0, 0)))
    pad_edges = jnp.full((2, E_P - E_RAW), N_RAW, jnp.int32)
    ei = jnp.concatenate([edge_index, pad_edges], axis=1)
    src3 = ei[0].reshape(NW, N_CHUNKS, CHUNK)
    dst3 = ei[1].reshape(NW, N_CHUNKS, CHUNK)

    cnt = _deg_kernel(src3, dst3)
    cnt_s = cnt[:, 0]
    cnt_d = cnt[:, 1]
    b0r = b0.reshape(1, F)
    b1r = b1.reshape(1, F)
    b2r = b2.reshape(1, F)

    t = _tc_first(feat_p, cnt_s, W0)
    p = _prop_kernel(t, src3, dst3)
    t = _tc_mid(p, cnt_s, cnt_d, b0r, W1)
    p = _prop_kernel(t, src3, dst3)
    t = _tc_mid(p, cnt_s, cnt_d, b1r, W2)
    p = _prop_kernel(t, src3, dst3)
    out = _tc_last(p, cnt_d, b2r)
    return out[:N_RAW]


# speedup vs baseline: 2.9265x; 2.9265x over previous
import functools

import jax
import jax.numpy as jnp
from jax import lax
from jax.experimental import pallas as pl
from jax.experimental.pallas import tpu as pltpu
from jax.experimental.pallas import tpu_sc as plsc

N_RAW = 10000
E_RAW = 320000
F = 128
N_P = 10240
E_P = 327680
NC, NS = 2, 16
NW = NC * NS
E_TILE = E_P // NW
CHUNK = 128
N_CHUNKS = E_TILE // CHUNK
ROWS_TILE = N_P // NS

_mesh = plsc.VectorSubcoreMesh(
    core_axis_name="c", subcore_axis_name="s", num_cores=NC, num_subcores=NS
)


def _zero_fill(ref, n_rows, n_cols):
    zero = jnp.zeros((16,), jnp.float32)

    def body(i, carry):
        for k in range(n_cols // 16):
            ref[i, pl.ds(k * 16, 16)] = zero
        return carry

    lax.fori_loop(0, n_rows, body, 0)


def _deg_body(src_hbm, dst_hbm, out_hbm, idx_s, idx_d, cnt_s, cnt_d,
              red_v, stage):
    c = lax.axis_index("c")
    s = lax.axis_index("s")
    wid = c * NS + s
    pltpu.sync_copy(src_hbm.at[wid], idx_s)
    pltpu.sync_copy(dst_hbm.at[wid], idx_d)
    zero = jnp.zeros((16,), jnp.float32)
    one = jnp.ones((16,), jnp.float32)

    def zboth(i, carry):
        cnt_s[pl.ds(i * 16, 16)] = zero
        cnt_d[pl.ds(i * 16, 16)] = zero
        return carry

    lax.fori_loop(0, N_P // 16, zboth, 0)

    def chunk(j, carry):
        for k in range(CHUNK // 16):
            iv_s = idx_s[j, pl.ds(k * 16, 16)]
            plsc.addupdate_scatter(cnt_s, [iv_s], one)
            iv_d = idx_d[j, pl.ds(k * 16, 16)]
            plsc.addupdate_scatter(cnt_d, [iv_d], one)
        return carry

    lax.fori_loop(0, N_CHUNKS, chunk, 0)
    pltpu.sync_copy(cnt_s, stage.at[s, 0])
    pltpu.sync_copy(cnt_d, stage.at[s, 1])
    plsc.subcore_barrier()
    base = s * ROWS_TILE
    for which in range(2):
        for t in range(NS):
            pltpu.sync_copy(stage.at[t, which, pl.ds(base, ROWS_TILE)],
                            red_v.at[t])

        def red(g, carry):
            acc = red_v[0, pl.ds(g * 16, 16)]
            for t in range(1, NS):
                acc = acc + red_v[t, pl.ds(g * 16, 16)]
            out_row = cnt_s if which == 0 else cnt_d
            out_row[pl.ds(g * 16, 16)] = acc
            return carry

        lax.fori_loop(0, ROWS_TILE // 16, red, 0)
        dst_ref = cnt_s if which == 0 else cnt_d
        pltpu.sync_copy(dst_ref.at[pl.ds(0, ROWS_TILE)],
                        out_hbm.at[c, which, pl.ds(base, ROWS_TILE)])


def _prop_body(table_hbm, src_hbm, dst_hbm, out_hbm, idx_s, idx_d, rows_v, agg):
    c = lax.axis_index("c")
    s = lax.axis_index("s")
    wid = c * NS + s
    pltpu.sync_copy(src_hbm.at[wid], idx_s)
    pltpu.sync_copy(dst_hbm.at[wid], idx_d)
    _zero_fill(rows_v, CHUNK, F)
    for r in range(ROWS_TILE // CHUNK):
        base = s * ROWS_TILE + r * CHUNK
        pltpu.sync_copy(rows_v, agg.at[pl.ds(base, CHUNK)])
    plsc.subcore_barrier()

    def chunk(j, carry):
        pltpu.sync_copy(table_hbm.at[idx_s.at[j]], rows_v)
        pltpu.sync_copy(rows_v, agg.at[idx_d.at[j]], add=True)
        return carry

    lax.fori_loop(0, N_CHUNKS, chunk, 0)
    plsc.subcore_barrier()
    pltpu.sync_copy(agg.at[pl.ds(s * ROWS_TILE, ROWS_TILE)],
                    out_hbm.at[c, pl.ds(s * ROWS_TILE, ROWS_TILE)])


_DEG_SCRATCH = [
    pltpu.VMEM((N_CHUNKS, CHUNK), jnp.int32),
    pltpu.VMEM((N_CHUNKS, CHUNK), jnp.int32),
    pltpu.VMEM((N_P,), jnp.float32),
    pltpu.VMEM((N_P,), jnp.float32),
    pltpu.VMEM((NS, ROWS_TILE), jnp.float32),
    pltpu.VMEM_SHARED((NS, 2, N_P), jnp.float32),
]
_PROP_SCRATCH = [
    pltpu.VMEM((N_CHUNKS, CHUNK), jnp.int32),
    pltpu.VMEM((N_CHUNKS, CHUNK), jnp.int32),
    pltpu.VMEM((CHUNK, F), jnp.float32),
    pltpu.VMEM_SHARED((N_P, F), jnp.float32),
]

_deg_kernel = pl.kernel(
    _deg_body,
    out_type=jax.ShapeDtypeStruct((NC, 2, N_P), jnp.float32),
    mesh=_mesh, scratch_types=_DEG_SCRATCH,
    compiler_params=pltpu.CompilerParams(needs_layout_passes=False))

_prop_kernel = pl.kernel(
    _prop_body,
    out_type=jax.ShapeDtypeStruct((NC, N_P, F), jnp.float32),
    mesh=_mesh, scratch_types=_PROP_SCRATCH)


def _deg_inv(cnt_pair):
    cnt = cnt_pair[0] + cnt_pair[1]
    return lax.rsqrt(jnp.maximum(cnt, 1.0))


def _tc_first_body(feat_ref, cs_ref, w_ref, out_ref):
    dsrc = _deg_inv(cs_ref[...])
    out_ref[...] = jnp.dot(feat_ref[...] * dsrc, w_ref[...],
                           preferred_element_type=jnp.float32)


def _tc_mid_body(p_ref, cs_ref, cd_ref, b_ref, w_ref, out_ref):
    p = p_ref[0] + p_ref[1]
    ddst = _deg_inv(cd_ref[...])
    h = jnp.maximum(p * ddst + b_ref[...], 0.0)
    dsrc = _deg_inv(cs_ref[...])
    out_ref[...] = jnp.dot(h * dsrc, w_ref[...],
                           preferred_element_type=jnp.float32)


def _tc_last_body(p_ref, cd_ref, b_ref, out_ref):
    p = p_ref[0] + p_ref[1]
    ddst = _deg_inv(cd_ref[...])
    out_ref[...] = p * ddst + b_ref[...]


_BLK = 1024
_GRID = N_P // _BLK

_feat_spec = pl.BlockSpec((_BLK, F), lambda i: (i, 0))
_cnt_spec = pl.BlockSpec((2, _BLK, 1), lambda i: (0, i, 0))
_p_spec = pl.BlockSpec((2, _BLK, F), lambda i: (0, i, 0))
_w_spec = pl.BlockSpec((F, F), lambda i: (0, 0))
_b_spec = pl.BlockSpec((1, F), lambda i: (0, 0))
_out_spec = pl.BlockSpec((_BLK, F), lambda i: (i, 0))
_out_shape = jax.ShapeDtypeStruct((N_P, F), jnp.float32)

_tc_first = pl.pallas_call(
    _tc_first_body, grid=(_GRID,),
    in_specs=[_feat_spec, _cnt_spec, _w_spec],
    out_specs=_out_spec, out_shape=_out_shape)

_tc_mid = pl.pallas_call(
    _tc_mid_body, grid=(_GRID,),
    in_specs=[_p_spec, _cnt_spec, _cnt_spec, _b_spec, _w_spec],
    out_specs=_out_spec, out_shape=_out_shape)

_tc_last = pl.pallas_call(
    _tc_last_body, grid=(_GRID,),
    in_specs=[_p_spec, _cnt_spec, _b_spec],
    out_specs=_out_spec, out_shape=_out_shape)


def kernel(features, edge_index, W0, b0, W1, b1, W2, b2):
    feat_p = jnp.pad(features, ((0, N_P - N_RAW), (0, 0)))
    pad_edges = jnp.full((2, E_P - E_RAW), N_RAW, jnp.int32)
    ei = jnp.concatenate([edge_index, pad_edges], axis=1)
    src3 = ei[0].reshape(NW, N_CHUNKS, CHUNK)
    dst3 = ei[1].reshape(NW, N_CHUNKS, CHUNK)

    cnt = _deg_kernel(src3, dst3)
    cnt_s = cnt[:, 0].reshape(NC, N_P, 1)
    cnt_d = cnt[:, 1].reshape(NC, N_P, 1)
    b0r = b0.reshape(1, F)
    b1r = b1.reshape(1, F)
    b2r = b2.reshape(1, F)

    t = _tc_first(feat_p, cnt_s, W0)
    p = _prop_kernel(t, src3, dst3)
    t = _tc_mid(p, cnt_s, cnt_d, b0r, W1)
    p = _prop_kernel(t, src3, dst3)
    t = _tc_mid(p, cnt_s, cnt_d, b1r, W2)
    p = _prop_kernel(t, src3, dst3)
    out = _tc_last(p, cnt_d, b2r)
    return out[:N_RAW]
